# baseline (speedup 1.0000x reference)
import functools

import jax
import jax.numpy as jnp
from jax import lax
from jax.experimental import pallas as pl
from jax.experimental.pallas import tpu as pltpu
from jax.experimental.pallas import tpu_sc as plsc

_NC = 2
_NS = 16
_NW = _NC * _NS
_CHUNK = 128
_PASS = 40
_PAD = 8


def _pad_edges(idx, n):
    e = idx.shape[0]
    per_w = e // _NW
    k = -(-per_w // _CHUNK)
    k = -(-k // _PASS) * _PASS
    pad = k * _CHUNK - per_w
    pad_idx = n + (jnp.arange(_NW * pad, dtype=jnp.int32) % _PAD)
    two_d = jnp.concatenate(
        [idx.reshape(_NW, per_w), pad_idx.reshape(_NW, pad)], axis=1)
    return two_d.reshape(_NW, k, _CHUNK)


def _sc_mesh():
    return plsc.VectorSubcoreMesh(core_axis_name="c", subcore_axis_name="s",
                                  num_cores=_NC, num_subcores=_NS)


def _copy_rows_sharded(sid, src, dst, n):
    per = (n // _NS) // 8 * 8
    tail = n - per * _NS
    pltpu.sync_copy(src.at[pl.ds(sid * per, per)], dst.at[pl.ds(sid * per, per)])
    if tail:
        @pl.when(sid == 0)
        def _():
            pltpu.sync_copy(src.at[pl.ds(per * _NS, tail)],
                            dst.at[pl.ds(per * _NS, tail)])


def _deg_body(dstc_hbm, ones_hbm, zeros_hbm, out_hbm, didx, ones_v, deg_sp):
    cid = lax.axis_index("c")
    sid = lax.axis_index("s")
    np_ = zeros_hbm.shape[0]
    _copy_rows_sharded(sid, zeros_hbm, deg_sp, np_)
    wid = cid * _NS + sid
    pltpu.sync_copy(dstc_hbm.at[wid], didx)
    pltpu.sync_copy(ones_hbm, ones_v)
    plsc.subcore_barrier()

    def body(g, carry):
        pltpu.sync_copy(ones_v, deg_sp.at[didx.at[g]], add=True)
        return carry

    lax.fori_loop(0, dstc_hbm.shape[1], body, 0)
    plsc.subcore_barrier()
    _copy_rows_sharded(sid, deg_sp, out_hbm.at[cid], np_)


def _sc_deg(dst, n, h):
    dstc = _pad_edges(dst, n)
    np_ = n + _PAD
    k = pl.kernel(
        _deg_body,
        out_type=jax.ShapeDtypeStruct((_NC, np_, h), jnp.float32),
        mesh=_sc_mesh(),
        scratch_types=[
            pltpu.VMEM(dstc.shape[1:], jnp.int32),
            pltpu.VMEM((_CHUNK, h), jnp.float32),
            pltpu.VMEM_SHARED((np_, h), jnp.float32),
        ],
    )
    return k(dstc, jnp.ones((_CHUNK, h), jnp.float32),
             jnp.zeros((np_, h), jnp.float32))


def _agg_body(y_hbm, srcc_hbm, dstc_hbm, zeros_hbm, out_hbm,
              sidx, didx, rows0, rows1, acc_sp, sem_a, sem_b):
    cid = lax.axis_index("c")
    sid = lax.axis_index("s")
    np_ = y_hbm.shape[0]
    _copy_rows_sharded(sid, zeros_hbm, acc_sp, np_)
    plsc.subcore_barrier()
    wid = cid * _NS + sid
    npass = srcc_hbm.shape[1] // _PASS

    def pass_body(p, carry):
        pltpu.sync_copy(srcc_hbm.at[wid, pl.ds(p * _PASS, _PASS)], sidx)
        pltpu.sync_copy(dstc_hbm.at[wid, pl.ds(p * _PASS, _PASS)], didx)
        pltpu.async_copy(y_hbm.at[sidx.at[0]], rows0, sem_a)
        pltpu.async_copy(y_hbm.at[sidx.at[1]], rows1, sem_b)

        def body(k, c):
            g = 2 * k
            pltpu.make_async_copy(y_hbm.at[sidx.at[0]], rows0, sem_a).wait()
            pltpu.sync_copy(rows0, acc_sp.at[didx.at[g]], add=True)

            @pl.when(g + 2 < _PASS)
            def _():
                pltpu.async_copy(y_hbm.at[sidx.at[g + 2]], rows0, sem_a)

            pltpu.make_async_copy(y_hbm.at[sidx.at[1]], rows1, sem_b).wait()
            pltpu.sync_copy(rows1, acc_sp.at[didx.at[g + 1]], add=True)

            @pl.when(g + 3 < _PASS)
            def _():
                pltpu.async_copy(y_hbm.at[sidx.at[g + 3]], rows1, sem_b)

            return c

        lax.fori_loop(0, _PASS // 2, body, 0)
        return carry

    lax.fori_loop(0, npass, pass_body, 0)
    plsc.subcore_barrier()
    _copy_rows_sharded(sid, acc_sp, out_hbm.at[cid], np_)


def _sc_agg(y, src, dst, n, h):
    srcc = _pad_edges(src, n)
    dstc = _pad_edges(dst, n)
    assert srcc.shape[1] % _PASS == 0
    np_ = n + _PAD
    y_pad = jnp.pad(y, ((0, _PAD), (0, 0)))
    k = pl.kernel(
        _agg_body,
        out_type=jax.ShapeDtypeStruct((_NC, np_, h), jnp.float32),
        mesh=_sc_mesh(),
        scratch_types=[
            pltpu.VMEM((_PASS, _CHUNK), jnp.int32),
            pltpu.VMEM((_PASS, _CHUNK), jnp.int32),
            pltpu.VMEM((_CHUNK, h), jnp.float32),
            pltpu.VMEM((_CHUNK, h), jnp.float32),
            pltpu.VMEM_SHARED((np_, h), jnp.float32),
            pltpu.SemaphoreType.DMA,
            pltpu.SemaphoreType.DMA,
        ],
    )
    return k(y_pad, srcc, dstc, jnp.zeros((np_, h), jnp.float32))


_BLK = 1000


def _prep_body(dp_ref, x_ref, w_ref, dinv_ref, y_ref):
    deg = dp_ref[0, :, 0:1] + dp_ref[1, :, 0:1] + 1.0
    dinv = lax.rsqrt(jnp.maximum(deg, 1.0))
    xw = jnp.dot(x_ref[...], w_ref[...], preferred_element_type=jnp.float32, precision=lax.Precision.HIGHEST)
    dinv_ref[...] = dinv
    y_ref[...] = xw * dinv


def _tc_prep(deg_parts, x, w0):
    n, d = x.shape
    grid = n // _BLK
    return pl.pallas_call(
        _prep_body,
        grid=(grid,),
        in_specs=[
            pl.BlockSpec((_NC, _BLK, d), lambda i: (0, i, 0)),
            pl.BlockSpec((_BLK, d), lambda i: (i, 0)),
            pl.BlockSpec((d, d), lambda i: (0, 0)),
        ],
        out_specs=[
            pl.BlockSpec((_BLK, 1), lambda i: (i, 0)),
            pl.BlockSpec((_BLK, d), lambda i: (i, 0)),
        ],
        out_shape=[
            jax.ShapeDtypeStruct((n, 1), jnp.float32),
            jax.ShapeDtypeStruct((n, d), jnp.float32),
        ],
    )(deg_parts, x, w0)


def _layer_body(add_residual, want_y,
                p_ref, y_ref, h_ref, dinv_ref, b_ref, g_ref, beta_ref, w_ref,
                *out_refs):
    dinv = dinv_ref[...]
    t = (p_ref[0] + p_ref[1] + y_ref[...]) * dinv + b_ref[...]
    m = jnp.mean(t, axis=1, keepdims=True)
    v = jnp.mean((t - m) ** 2, axis=1, keepdims=True)
    t = (t - m) * lax.rsqrt(v + 1e-5) * g_ref[...] + beta_ref[...]
    t = jnp.maximum(t, 0.0)
    if add_residual:
        t = t + h_ref[...]
    out_refs[0][...] = t
    if want_y:
        out_refs[1][...] = jnp.dot(
            t, w_ref[...], preferred_element_type=jnp.float32,
            precision=lax.Precision.HIGHEST) * dinv


def _tc_layer(parts, y, h, dinv, b, g, beta, w_next, add_residual, want_y):
    n, d = y.shape
    grid = n // _BLK
    out_shape = [jax.ShapeDtypeStruct((n, d), jnp.float32)]
    out_specs = [pl.BlockSpec((_BLK, d), lambda i: (i, 0))]
    if want_y:
        out_shape.append(jax.ShapeDtypeStruct((n, d), jnp.float32))
        out_specs.append(pl.BlockSpec((_BLK, d), lambda i: (i, 0)))
    return pl.pallas_call(
        functools.partial(_layer_body, add_residual, want_y),
        grid=(grid,),
        in_specs=[
            pl.BlockSpec((_NC, _BLK, d), lambda i: (0, i, 0)),
            pl.BlockSpec((_BLK, d), lambda i: (i, 0)),
            pl.BlockSpec((_BLK, d), lambda i: (i, 0)),
            pl.BlockSpec((_BLK, 1), lambda i: (i, 0)),
            pl.BlockSpec((1, d), lambda i: (0, 0)),
            pl.BlockSpec((1, d), lambda i: (0, 0)),
            pl.BlockSpec((1, d), lambda i: (0, 0)),
            pl.BlockSpec((d, d), lambda i: (0, 0)),
        ],
        out_specs=out_specs,
        out_shape=out_shape,
    )(parts, y, h, dinv, b, g, beta, w_next)


def _head_body(ngroups, h_ref, batch_ref, bn1g, bn1b, fc1w, fc1b,
               bn2g, bn2b, fc2w, fc2b, out_ref, pooled_acc, cnt_acc):
    i = pl.program_id(0)
    blk = h_ref.shape[0]

    @pl.when(i == 0)
    def _():
        pooled_acc[...] = jnp.zeros_like(pooled_acc)
        cnt_acc[...] = jnp.zeros_like(cnt_acc)

    onehot = (lax.broadcasted_iota(jnp.int32, (blk, ngroups), 1)
              == batch_ref[...]).astype(jnp.float32)
    pooled_acc[...] += lax.dot_general(
        onehot, h_ref[...], (((0,), (0,)), ((), ())),
        preferred_element_type=jnp.float32, precision=lax.Precision.HIGHEST)
    cnt_acc[...] += lax.dot_general(
        onehot, jnp.ones((blk, 1), jnp.float32), (((0,), (0,)), ((), ())),
        preferred_element_type=jnp.float32)

    @pl.when(i == pl.num_programs(0) - 1)
    def _():
        pooled = pooled_acc[...] / jnp.maximum(cnt_acc[...], 1.0)
        m = jnp.mean(pooled, axis=0, keepdims=True)
        v = jnp.mean((pooled - m) ** 2, axis=0, keepdims=True)
        t = (pooled - m) * lax.rsqrt(v + 1e-5) * bn1g[...] + bn1b[...]
        t = jnp.maximum(t, 0.0)
        t = jnp.dot(t, fc1w[...], preferred_element_type=jnp.float32,
                    precision=lax.Precision.HIGHEST) + fc1b[...]
        m2 = jnp.mean(t, axis=0, keepdims=True)
        v2 = jnp.mean((t - m2) ** 2, axis=0, keepdims=True)
        t = (t - m2) * lax.rsqrt(v2 + 1e-5) * bn2g[...] + bn2b[...]
        t = jnp.maximum(t, 0.0)
        out_ref[...] = jnp.dot(
            t, fc2w[...], preferred_element_type=jnp.float32,
            precision=lax.Precision.HIGHEST) + fc2b[...]


def _tc_head(h, batch2d, bn1_g, bn1_b, fc1_W, fc1_b, bn2_g, bn2_b,
             fc2_W, fc2_b, ngroups):
    n, d = h.shape
    dh = fc1_W.shape[1]
    grid = n // _BLK
    return pl.pallas_call(
        functools.partial(_head_body, ngroups),
        grid=(grid,),
        in_specs=[
            pl.BlockSpec((_BLK, d), lambda i: (i, 0)),
            pl.BlockSpec((_BLK, 1), lambda i: (i, 0)),
            pl.BlockSpec((1, d), lambda i: (0, 0)),
            pl.BlockSpec((1, d), lambda i: (0, 0)),
            pl.BlockSpec((d, dh), lambda i: (0, 0)),
            pl.BlockSpec((1, dh), lambda i: (0, 0)),
            pl.BlockSpec((1, dh), lambda i: (0, 0)),
            pl.BlockSpec((1, dh), lambda i: (0, 0)),
            pl.BlockSpec((dh, 1), lambda i: (0, 0)),
            pl.BlockSpec((1, 1), lambda i: (0, 0)),
        ],
        out_specs=pl.BlockSpec((ngroups, 1), lambda i: (0, 0)),
        out_shape=jax.ShapeDtypeStruct((ngroups, 1), jnp.float32),
        scratch_shapes=[
            pltpu.VMEM((ngroups, d), jnp.float32),
            pltpu.VMEM((ngroups, 1), jnp.float32),
        ],
        compiler_params=pltpu.CompilerParams(
            dimension_semantics=("arbitrary",)),
    )(h, batch2d, bn1_g, bn1_b, fc1_W, fc1_b, bn2_g, bn2_b, fc2_W, fc2_b)


def kernel(x, Ws, bs, ln_g, ln_b, bn1_g, bn1_b, fc1_W, fc1_b,
           bn2_g, bn2_b, fc2_W, fc2_b, edge_index, batch):
    n, d = x.shape
    nlayers = Ws.shape[0]
    ngroups = 64
    src = edge_index[0]
    dst = edge_index[1]

    deg_parts = _sc_deg(dst, n, d)
    dinv, y = _tc_prep(deg_parts, x, Ws[0])

    h = x
    for i in range(nlayers):
        parts = _sc_agg(y, src, dst, n, d)
        last = i == nlayers - 1
        w_next = Ws[i + 1] if not last else Ws[i]
        outs = _tc_layer(parts, y, h, dinv,
                         bs[i].reshape(1, d), ln_g[i].reshape(1, d),
                         ln_b[i].reshape(1, d), w_next,
                         add_residual=(i > 0), want_y=not last)
        if last:
            h = outs[0]
        else:
            h, y = outs

    return _tc_head(h, batch.reshape(n, 1).astype(jnp.int32),
                    bn1_g.reshape(1, d), bn1_b.reshape(1, d),
                    fc1_W, fc1_b.reshape(1, -1),
                    bn2_g.reshape(1, -1), bn2_b.reshape(1, -1),
                    fc2_W, fc2_b.reshape(1, 1), ngroups)


# speedup vs baseline: 1.0274x; 1.0274x over previous
import functools

import jax
import jax.numpy as jnp
from jax import lax
from jax.experimental import pallas as pl
from jax.experimental.pallas import tpu as pltpu
from jax.experimental.pallas import tpu_sc as plsc

_NC = 2
_NS = 16
_NW = _NC * _NS
_CHUNK = 128
_PASS = 40
_PAD = 8


def _pad_edges(idx, n):
    e = idx.shape[0]
    per_w = e // _NW
    k = -(-per_w // _CHUNK)
    k = -(-k // _PASS) * _PASS
    pad = k * _CHUNK - per_w
    pad_idx = n + (jnp.arange(_NW * pad, dtype=jnp.int32) % _PAD)
    two_d = jnp.concatenate(
        [idx.reshape(_NW, per_w), pad_idx.reshape(_NW, pad)], axis=1)
    return two_d.reshape(_NW, k, _CHUNK)


def _sc_mesh():
    return plsc.VectorSubcoreMesh(core_axis_name="c", subcore_axis_name="s",
                                  num_cores=_NC, num_subcores=_NS)


def _copy_rows_sharded(sid, src, dst, n):
    per = (n // _NS) // 8 * 8
    tail = n - per * _NS
    pltpu.sync_copy(src.at[pl.ds(sid * per, per)], dst.at[pl.ds(sid * per, per)])
    if tail:
        @pl.when(sid == 0)
        def _():
            pltpu.sync_copy(src.at[pl.ds(per * _NS, tail)],
                            dst.at[pl.ds(per * _NS, tail)])


def _deg_body(dstc_hbm, ones_hbm, zeros_hbm, out_hbm, didx, ones_v, deg_sp):
    cid = lax.axis_index("c")
    sid = lax.axis_index("s")
    np_ = zeros_hbm.shape[0]
    _copy_rows_sharded(sid, zeros_hbm, deg_sp, np_)
    wid = cid * _NS + sid
    pltpu.sync_copy(dstc_hbm.at[wid], didx)
    pltpu.sync_copy(ones_hbm, ones_v)
    plsc.subcore_barrier()

    def body(g, carry):
        pltpu.sync_copy(ones_v, deg_sp.at[didx.at[g]], add=True)
        return carry

    lax.fori_loop(0, dstc_hbm.shape[1], body, 0)
    plsc.subcore_barrier()
    _copy_rows_sharded(sid, deg_sp, out_hbm.at[cid], np_)


def _sc_deg(dst, n, h):
    dstc = _pad_edges(dst, n)
    np_ = n + _PAD
    k = pl.kernel(
        _deg_body,
        out_type=jax.ShapeDtypeStruct((_NC, np_, h), jnp.float32),
        mesh=_sc_mesh(),
        scratch_types=[
            pltpu.VMEM(dstc.shape[1:], jnp.int32),
            pltpu.VMEM((_CHUNK, h), jnp.float32),
            pltpu.VMEM_SHARED((np_, h), jnp.float32),
        ],
    )
    return k(dstc, jnp.ones((_CHUNK, h), jnp.float32),
             jnp.zeros((np_, h), jnp.float32))


def _agg_body(y_hbm, srcc_hbm, dstc_hbm, zeros_hbm, out_hbm,
              sidx, didx, rows0, rows1, acc_sp, sem_a, sem_b):
    cid = lax.axis_index("c")
    sid = lax.axis_index("s")
    np_ = y_hbm.shape[0]
    _copy_rows_sharded(sid, zeros_hbm, acc_sp, np_)
    plsc.subcore_barrier()
    wid = cid * _NS + sid
    npass = srcc_hbm.shape[1] // _PASS

    def pass_body(p, carry):
        pltpu.sync_copy(srcc_hbm.at[wid, pl.ds(p * _PASS, _PASS)], sidx)
        pltpu.sync_copy(dstc_hbm.at[wid, pl.ds(p * _PASS, _PASS)], didx)
        pltpu.async_copy(y_hbm.at[sidx.at[0]], rows0, sem_a)
        pltpu.async_copy(y_hbm.at[sidx.at[1]], rows1, sem_b)

        def body(k, c):
            g = 2 * k
            pltpu.make_async_copy(y_hbm.at[sidx.at[0]], rows0, sem_a).wait()
            pltpu.sync_copy(rows0, acc_sp.at[didx.at[g]], add=True)

            @pl.when(g + 2 < _PASS)
            def _():
                pltpu.async_copy(y_hbm.at[sidx.at[g + 2]], rows0, sem_a)

            pltpu.make_async_copy(y_hbm.at[sidx.at[1]], rows1, sem_b).wait()
            pltpu.sync_copy(rows1, acc_sp.at[didx.at[g + 1]], add=True)

            @pl.when(g + 3 < _PASS)
            def _():
                pltpu.async_copy(y_hbm.at[sidx.at[g + 3]], rows1, sem_b)

            return c

        lax.fori_loop(0, _PASS // 2, body, 0)
        return carry

    lax.fori_loop(0, npass, pass_body, 0)
    plsc.subcore_barrier()
    _copy_rows_sharded(sid, acc_sp, out_hbm.at[cid], np_)


def _sc_agg(y_pad, src, dst, n, h):
    srcc = _pad_edges(src, n)
    dstc = _pad_edges(dst, n)
    assert srcc.shape[1] % _PASS == 0
    np_ = n + _PAD
    k = pl.kernel(
        _agg_body,
        out_type=jax.ShapeDtypeStruct((_NC, np_, h), jnp.float32),
        mesh=_sc_mesh(),
        scratch_types=[
            pltpu.VMEM((_PASS, _CHUNK), jnp.int32),
            pltpu.VMEM((_PASS, _CHUNK), jnp.int32),
            pltpu.VMEM((_CHUNK, h), jnp.float32),
            pltpu.VMEM((_CHUNK, h), jnp.float32),
            pltpu.VMEM_SHARED((np_, h), jnp.float32),
            pltpu.SemaphoreType.DMA,
            pltpu.SemaphoreType.DMA,
        ],
    )
    return k(y_pad, srcc, dstc, jnp.zeros((np_, h), jnp.float32))


_BLK = 1112


def _mm_body(x_ref, w_ref, o_ref):
    o_ref[...] = jnp.dot(x_ref[...], w_ref[...],
                         preferred_element_type=jnp.float32,
                         precision=lax.Precision.HIGHEST)


def _tc_matmul(x, w):
    np_, d = x.shape
    return pl.pallas_call(
        _mm_body,
        grid=(np_ // _BLK,),
        in_specs=[
            pl.BlockSpec((_BLK, d), lambda i: (i, 0)),
            pl.BlockSpec((d, d), lambda i: (0, 0)),
        ],
        out_specs=pl.BlockSpec((_BLK, d), lambda i: (i, 0)),
        out_shape=jax.ShapeDtypeStruct((np_, d), jnp.float32),
    )(x, w)


def _scale_body(dp_ref, xw_ref, dinv_ref, y_ref):
    deg = dp_ref[0, :, 0:1] + dp_ref[1, :, 0:1] + 1.0
    dinv = lax.rsqrt(jnp.maximum(deg, 1.0))
    dinv_ref[...] = dinv
    y_ref[...] = xw_ref[...] * dinv


def _tc_scale(deg_parts, xw):
    np_, d = xw.shape
    return pl.pallas_call(
        _scale_body,
        grid=(np_ // _BLK,),
        in_specs=[
            pl.BlockSpec((_NC, _BLK, d), lambda i: (0, i, 0)),
            pl.BlockSpec((_BLK, d), lambda i: (i, 0)),
        ],
        out_specs=[
            pl.BlockSpec((_BLK, 1), lambda i: (i, 0)),
            pl.BlockSpec((_BLK, d), lambda i: (i, 0)),
        ],
        out_shape=[
            jax.ShapeDtypeStruct((np_, 1), jnp.float32),
            jax.ShapeDtypeStruct((np_, d), jnp.float32),
        ],
    )(deg_parts, xw)


def _layer_body(add_residual, want_y,
                p_ref, y_ref, h_ref, dinv_ref, b_ref, g_ref, beta_ref, w_ref,
                *out_refs):
    dinv = dinv_ref[...]
    t = (p_ref[0] + p_ref[1] + y_ref[...]) * dinv + b_ref[...]
    m = jnp.mean(t, axis=1, keepdims=True)
    v = jnp.mean((t - m) ** 2, axis=1, keepdims=True)
    t = (t - m) * lax.rsqrt(v + 1e-5) * g_ref[...] + beta_ref[...]
    t = jnp.maximum(t, 0.0)
    if add_residual:
        t = t + h_ref[...]
    out_refs[0][...] = t
    if want_y:
        out_refs[1][...] = jnp.dot(
            t, w_ref[...], preferred_element_type=jnp.float32,
            precision=lax.Precision.HIGHEST) * dinv


def _tc_layer(parts, y, h, dinv, b, g, beta, w_next, add_residual, want_y):
    np_, d = y.shape
    grid = np_ // _BLK
    out_shape = [jax.ShapeDtypeStruct((np_, d), jnp.float32)]
    out_specs = [pl.BlockSpec((_BLK, d), lambda i: (i, 0))]
    if want_y:
        out_shape.append(jax.ShapeDtypeStruct((np_, d), jnp.float32))
        out_specs.append(pl.BlockSpec((_BLK, d), lambda i: (i, 0)))
    return pl.pallas_call(
        functools.partial(_layer_body, add_residual, want_y),
        grid=(grid,),
        in_specs=[
            pl.BlockSpec((_NC, _BLK, d), lambda i: (0, i, 0)),
            pl.BlockSpec((_BLK, d), lambda i: (i, 0)),
            pl.BlockSpec((_BLK, d), lambda i: (i, 0)),
            pl.BlockSpec((_BLK, 1), lambda i: (i, 0)),
            pl.BlockSpec((1, d), lambda i: (0, 0)),
            pl.BlockSpec((1, d), lambda i: (0, 0)),
            pl.BlockSpec((1, d), lambda i: (0, 0)),
            pl.BlockSpec((d, d), lambda i: (0, 0)),
        ],
        out_specs=out_specs,
        out_shape=out_shape,
    )(parts, y, h, dinv, b, g, beta, w_next)


def _head_body(ngroups, h_ref, batch_ref, bn1g, bn1b, fc1w, fc1b,
               bn2g, bn2b, fc2w, fc2b, out_ref, pooled_acc, cnt_acc):
    i = pl.program_id(0)
    blk = h_ref.shape[0]

    @pl.when(i == 0)
    def _():
        pooled_acc[...] = jnp.zeros_like(pooled_acc)
        cnt_acc[...] = jnp.zeros_like(cnt_acc)

    onehot = (lax.broadcasted_iota(jnp.int32, (blk, ngroups), 1)
              == batch_ref[...]).astype(jnp.float32)
    pooled_acc[...] += lax.dot_general(
        onehot, h_ref[...], (((0,), (0,)), ((), ())),
        preferred_element_type=jnp.float32, precision=lax.Precision.HIGHEST)
    cnt_acc[...] += lax.dot_general(
        onehot, jnp.ones((blk, 1), jnp.float32), (((0,), (0,)), ((), ())),
        preferred_element_type=jnp.float32)

    @pl.when(i == pl.num_programs(0) - 1)
    def _():
        pooled = pooled_acc[...] / jnp.maximum(cnt_acc[...], 1.0)
        m = jnp.mean(pooled, axis=0, keepdims=True)
        v = jnp.mean((pooled - m) ** 2, axis=0, keepdims=True)
        t = (pooled - m) * lax.rsqrt(v + 1e-5) * bn1g[...] + bn1b[...]
        t = jnp.maximum(t, 0.0)
        t = jnp.dot(t, fc1w[...], preferred_element_type=jnp.float32,
                    precision=lax.Precision.HIGHEST) + fc1b[...]
        m2 = jnp.mean(t, axis=0, keepdims=True)
        v2 = jnp.mean((t - m2) ** 2, axis=0, keepdims=True)
        t = (t - m2) * lax.rsqrt(v2 + 1e-5) * bn2g[...] + bn2b[...]
        t = jnp.maximum(t, 0.0)
        out_ref[...] = jnp.dot(
            t, fc2w[...], preferred_element_type=jnp.float32,
            precision=lax.Precision.HIGHEST) + fc2b[...]


def _tc_head(h, batch2d, bn1_g, bn1_b, fc1_W, fc1_b, bn2_g, bn2_b,
             fc2_W, fc2_b, ngroups):
    n, d = h.shape
    dh = fc1_W.shape[1]
    grid = n // _BLK
    return pl.pallas_call(
        functools.partial(_head_body, ngroups),
        grid=(grid,),
        in_specs=[
            pl.BlockSpec((_BLK, d), lambda i: (i, 0)),
            pl.BlockSpec((_BLK, 1), lambda i: (i, 0)),
            pl.BlockSpec((1, d), lambda i: (0, 0)),
            pl.BlockSpec((1, d), lambda i: (0, 0)),
            pl.BlockSpec((d, dh), lambda i: (0, 0)),
            pl.BlockSpec((1, dh), lambda i: (0, 0)),
            pl.BlockSpec((1, dh), lambda i: (0, 0)),
            pl.BlockSpec((1, dh), lambda i: (0, 0)),
            pl.BlockSpec((dh, 1), lambda i: (0, 0)),
            pl.BlockSpec((1, 1), lambda i: (0, 0)),
        ],
        out_specs=pl.BlockSpec((ngroups, 1), lambda i: (0, 0)),
        out_shape=jax.ShapeDtypeStruct((ngroups, 1), jnp.float32),
        scratch_shapes=[
            pltpu.VMEM((ngroups, d), jnp.float32),
            pltpu.VMEM((ngroups, 1), jnp.float32),
        ],
        compiler_params=pltpu.CompilerParams(
            dimension_semantics=("arbitrary",)),
    )(h, batch2d, bn1_g, bn1_b, fc1_W, fc1_b, bn2_g, bn2_b, fc2_W, fc2_b)


def kernel(x, Ws, bs, ln_g, ln_b, bn1_g, bn1_b, fc1_W, fc1_b,
           bn2_g, bn2_b, fc2_W, fc2_b, edge_index, batch):
    n, d = x.shape
    nlayers = Ws.shape[0]
    ngroups = 64
    src = edge_index[0]
    dst = edge_index[1]

    x_pad = jnp.pad(x, ((0, _PAD), (0, 0)))
    batch_pad = jnp.pad(batch.astype(jnp.int32).reshape(n, 1),
                        ((0, _PAD), (0, 0)), constant_values=-1)

    deg_parts = _sc_deg(dst, n, d)
    xw0 = _tc_matmul(x_pad, Ws[0])
    dinv, y = _tc_scale(deg_parts, xw0)

    h = x_pad
    for i in range(nlayers):
        parts = _sc_agg(y, src, dst, n, d)
        last = i == nlayers - 1
        w_next = Ws[i + 1] if not last else Ws[i]
        outs = _tc_layer(parts, y, h, dinv,
                         bs[i].reshape(1, d), ln_g[i].reshape(1, d),
                         ln_b[i].reshape(1, d), w_next,
                         add_residual=(i > 0), want_y=not last)
        if last:
            h = outs[0]
        else:
            h, y = outs

    return _tc_head(h, batch_pad,
                    bn1_g.reshape(1, d), bn1_b.reshape(1, d),
                    fc1_W, fc1_b.reshape(1, -1),
                    bn2_g.reshape(1, -1), bn2_b.reshape(1, -1),
                    fc2_W, fc2_b.reshape(1, 1), ngroups)
